# baseline (speedup 1.0000x reference)
import functools

import jax
import jax.numpy as jnp
from jax import lax
from jax.experimental import pallas as pl
from jax.experimental.pallas import tpu as pltpu
from jax.experimental.pallas import tpu_sc as plsc

EMB = 64
LANES = 16
NCORES = 2
NWORK = 32
ENT = 1000000
BATCH = 16384
NQ = 2 * BATCH
NBLK = 7813
BLKS_PW = 246
EPW = BLKS_PW * 128
QCAP = 1552
SEGCAP = 192
NSUP = 16
DUMP = NQ
SROWS = 128


def _iota16():
    return lax.iota(jnp.int32, LANES)


def _bcast(vec, lane):
    idx = jnp.full((LANES,), 0, jnp.int32) + lane
    return vec.at[idx].get(mode="promise_in_bounds")


def _extract_body(ent_t, proj_t, h, t, staged,
                  h_v, t_v, qe, qp, qe2, qp2, ae, ap,
                  bufw_a, bufp_a, bufw_b, bufp_b,
                  srow, posb, scnt, sem_a, sem_b, sem_f):
    wid = lax.axis_index("s") * NCORES + lax.axis_index("c")
    b0 = wid * BLKS_PW
    e0 = b0 * 128
    e1 = jnp.minimum(e0 + EPW, ENT)
    it16 = _iota16()

    pltpu.sync_copy(h, h_v)
    pltpu.sync_copy(t, t_v)

    def scan_src(src_ref, pos_base, cnt0):
        def body(i, cnt):
            e = src_ref[pl.ds(i * LANES, LANES)]
            pos = it16 + (i * LANES + pos_base)
            m = (e >= e0) & (e < e1)
            plsc.store_compressed(qe.at[pl.ds(cnt, LANES)], e, mask=m)
            plsc.store_compressed(qp.at[pl.ds(cnt, LANES)], pos, mask=m)
            cnt = cnt + plsc.all_reduce_population_count(m)[0]
            return jnp.minimum(cnt, QCAP - LANES)
        return lax.fori_loop(0, BATCH // LANES, body, cnt0)

    cntq = scan_src(h_v, 0, jnp.int32(0))
    cntq = scan_src(t_v, BATCH, cntq)

    for s in range(NSUP):
        lo = e0 + s * (16 * 128)
        hi = jnp.minimum(lo + 16 * 128, e1)

        def seg_body(i, c2, lo=lo, hi=hi, s=s):
            e = qe[pl.ds(i * LANES, LANES)]
            p = qp[pl.ds(i * LANES, LANES)]
            valid = (i * LANES + it16) < cntq
            m = valid & (e >= lo) & (e < hi)
            plsc.store_compressed(qe2.at[pl.ds(s * SEGCAP + c2, LANES)], e, mask=m)
            plsc.store_compressed(qp2.at[pl.ds(s * SEGCAP + c2, LANES)], p, mask=m)
            c2 = c2 + plsc.all_reduce_population_count(m)[0]
            return jnp.minimum(c2, SEGCAP - LANES)

        c2 = lax.fori_loop(0, QCAP // LANES, seg_body, jnp.int32(0))
        scnt[s] = c2

    blk_lim = jnp.minimum(b0 + BLKS_PW, NBLK)

    def issue(b, bufw, bufp):
        sem = sem_a if bufw is bufw_a else sem_b

        @pl.when(b < blk_lim)
        def _():
            c0 = b * 128
            pltpu.async_copy(ent_t.at[:, pl.ds(c0, 128)], bufw, sem)
            pltpu.async_copy(proj_t.at[:, pl.ds(c0, 128)], bufp, sem)

    def wait_set(b, bufw, bufp, sem):
        @pl.when(b < blk_lim)
        def _():
            pltpu.make_async_copy(ent_t.at[:, pl.ds(0, 128)], bufw, sem).wait()
            pltpu.make_async_copy(proj_t.at[:, pl.ds(0, 128)], bufp, sem).wait()

    def gather_block(b, bufw, bufp, iters, cmask, slot):
        s = (b - b0) >> 4
        sbase = s * SEGCAP
        slim = scnt[s]

        def find(i, cb):
            e = qe2[pl.ds(sbase + i * LANES, LANES)]
            p = qp2[pl.ds(sbase + i * LANES, LANES)]
            valid = (i * LANES + it16) < slim
            m = valid & ((e >> 7) == b) & cmask
            plsc.store_compressed(ae.at[pl.ds(cb, LANES)], e, mask=m)
            plsc.store_compressed(ap.at[pl.ds(cb, LANES)], p, mask=m)
            return cb + plsc.all_reduce_population_count(m)[0]

        cb = jnp.minimum(lax.fori_loop(0, iters, find, jnp.int32(0)), 64)

        def one_query(i, sl):
            k16 = (i // LANES) * LANES
            lane = i - k16
            esub = ae[pl.ds(k16, LANES)]
            psub = ap[pl.ds(k16, LANES)]
            cvec = _bcast(esub, lane) & 127
            pvec = _bcast(psub, lane)
            sl_hi = sl // 64
            sl_lo = sl - sl_hi * 64
            plsc.store_scatter(posb, [jnp.full((LANES,), 0, jnp.int32) + sl_hi,
                                      jnp.full((LANES,), 0, jnp.int32) + sl_lo],
                               pvec, mask=it16 == 0)
            slv = jnp.full((LANES,), 0, jnp.int32) + sl
            for k in range(4):
                rows = it16 + (k * LANES)
                wv = plsc.load_gather(bufw, [rows, cvec])
                pv = plsc.load_gather(bufp, [rows, cvec])
                plsc.store_scatter(srow, [slv, rows], wv)
                plsc.store_scatter(srow, [slv, rows + EMB], pv)
            return sl + 1

        return lax.fori_loop(0, cb, one_query, slot)

    def flush64(slot):
        def do(sl):
            pltpu.async_copy(srow.at[pl.ds(0, 64)], staged.at[posb.at[0]], sem_f).wait()
            rem = sl - 64
            for k in range(4):
                posb[0, pl.ds(k * LANES, LANES)] = posb[1, pl.ds(k * LANES, LANES)]

            def mv(i, _):
                src = jnp.full((LANES,), 64, jnp.int32) + i
                dst = jnp.full((LANES,), 0, jnp.int32) + i
                for k in range(8):
                    cols = _iota16() + (k * LANES)
                    v = plsc.load_gather(srow, [src, cols])
                    plsc.store_scatter(srow, [dst, cols], v)
                return 0
            lax.fori_loop(0, rem, mv, 0)
            return rem
        return lax.cond(slot >= 64, do, lambda sl: sl, slot)

    issue(b0, bufw_a, bufp_a)

    def pair_body(k2, slot):
        a = b0 + 2 * k2
        issue(a + 1, bufw_b, bufp_b)
        wait_set(a, bufw_a, bufp_a, sem_a)
        slot = lax.cond(a < blk_lim,
                        lambda sl: gather_block(a, bufw_a, bufp_a,
                                                SEGCAP // LANES, it16 >= 0, sl),
                        lambda sl: sl, slot)
        slot = flush64(slot)
        issue(a + 2, bufw_a, bufp_a)
        wait_set(a + 1, bufw_b, bufp_b, sem_b)
        slot = lax.cond(a + 1 < blk_lim,
                        lambda sl: gather_block(a + 1, bufw_b, bufp_b,
                                                SEGCAP // LANES, it16 >= 0, sl),
                        lambda sl: sl, slot)
        return flush64(slot)

    slot = lax.fori_loop(0, BLKS_PW // 2, pair_body, jnp.int32(0))

    for k in range(4):
        lanes = it16 + (k * LANES)
        cur = posb[0, pl.ds(k * LANES, LANES)]
        posb[0, pl.ds(k * LANES, LANES)] = jnp.where(
            lanes < slot, cur, DUMP + lanes)
    pltpu.async_copy(srow.at[pl.ds(0, 64)], staged.at[posb.at[0]], sem_f).wait()


def _score_body(staged, relc, r, out,
                hbuf0, tbuf0, rbuf0, ridx0, hbuf1, tbuf1, rbuf1, ridx1,
                bscr, out_v, sem0, sem1):
    wid = lax.axis_index("s") * NCORES + lax.axis_index("c")
    rows_pw = out_v.shape[0]
    base = wid * rows_pw
    it16 = _iota16()
    nch = rows_pw // 128
    sets = [(hbuf0, tbuf0, rbuf0, ridx0, sem0),
            (hbuf1, tbuf1, rbuf1, ridx1, sem1)]

    def issue(c):
        hbuf, tbuf, rbuf, ridx, sem = sets[c % 2]
        off = base + c * 128
        pltpu.sync_copy(r.at[pl.ds(off, 128)], ridx)
        pltpu.async_copy(staged.at[pl.ds(off, 128)], hbuf, sem)
        pltpu.async_copy(staged.at[pl.ds(BATCH + off, 128)], tbuf, sem)
        pltpu.async_copy(relc.at[ridx], rbuf, sem)

    def wait(c):
        hbuf, tbuf, rbuf, ridx, sem = sets[c % 2]
        off = base + c * 128
        pltpu.make_async_copy(staged.at[pl.ds(off, 128)], hbuf, sem).wait()
        pltpu.make_async_copy(staged.at[pl.ds(off, 128)], tbuf, sem).wait()
        pltpu.make_async_copy(staged.at[pl.ds(off, 128)], rbuf, sem).wait()

    issue(0)
    for c in range(nch):
        wait(c)
        if c + 1 < nch:
            issue(c + 1)
        hbuf, tbuf, rbuf, ridx, _ = sets[c % 2]

        def group_body(g, carry, c=c, hbuf=hbuf, tbuf=tbuf, rbuf=rbuf):
            rows = it16 + g * LANES
            sh = [jnp.zeros((LANES,), jnp.float32) for _ in range(4)]
            st = [jnp.zeros((LANES,), jnp.float32) for _ in range(4)]
            for j in range(EMB):
                cj = jnp.full((LANES,), j, jnp.int32)
                he_j = plsc.load_gather(hbuf, [rows, cj])
                hp_j = plsc.load_gather(hbuf, [rows, cj + EMB])
                te_j = plsc.load_gather(tbuf, [rows, cj])
                tp_j = plsc.load_gather(tbuf, [rows, cj + EMB])
                re_j = plsc.load_gather(rbuf, [rows, cj])
                sh[j % 4] = sh[j % 4] + he_j * hp_j
                st[j % 4] = st[j % 4] + te_j * tp_j
                bscr[j] = he_j - te_j + re_j
            a = (sh[0] + sh[1]) + (sh[2] + sh[3]) \
                - ((st[0] + st[1]) + (st[2] + st[3]))
            acc = [jnp.zeros((LANES,), jnp.float32) for _ in range(4)]
            for j in range(EMB):
                cj = jnp.full((LANES,), j, jnp.int32)
                rp_j = plsc.load_gather(rbuf, [rows, cj + EMB])
                acc[j % 4] = acc[j % 4] + jnp.abs(bscr[j] + a * rp_j)
            score = (acc[0] + acc[1]) + (acc[2] + acc[3])
            out_v[pl.ds(c * 128 + g * LANES, LANES)] = score
            return carry

        lax.fori_loop(0, 8, group_body, 0)

    pltpu.sync_copy(out_v, out.at[wid])


def kernel(ent_w, rel_w, ent_proj_w, rel_proj_w, h, t, r):
    mesh = plsc.VectorSubcoreMesh(core_axis_name="c", subcore_axis_name="s")
    cp = pltpu.CompilerParams(use_tc_tiling_on_sc=True,
                              needs_layout_passes=False)

    extract = pl.kernel(
        _extract_body,
        out_type=jax.ShapeDtypeStruct((NQ + 64, 128), jnp.float32),
        mesh=mesh,
        compiler_params=cp,
        scratch_types=[
            pltpu.VMEM((BATCH,), jnp.int32),
            pltpu.VMEM((BATCH,), jnp.int32),
            pltpu.VMEM((QCAP,), jnp.int32),
            pltpu.VMEM((QCAP,), jnp.int32),
            pltpu.VMEM((NSUP * SEGCAP,), jnp.int32),
            pltpu.VMEM((NSUP * SEGCAP,), jnp.int32),
            pltpu.VMEM((80,), jnp.int32),
            pltpu.VMEM((80,), jnp.int32),
            pltpu.VMEM((EMB, 128), jnp.float32),
            pltpu.VMEM((EMB, 128), jnp.float32),
            pltpu.VMEM((EMB, 128), jnp.float32),
            pltpu.VMEM((EMB, 128), jnp.float32),
            pltpu.VMEM((SROWS, 128), jnp.float32),
            pltpu.VMEM((2, 64), jnp.int32),
            pltpu.SMEM((NSUP,), jnp.int32),
            pltpu.SemaphoreType.DMA,
            pltpu.SemaphoreType.DMA,
            pltpu.SemaphoreType.DMA,
        ],
    )

    score = pl.kernel(
        _score_body,
        out_type=jax.ShapeDtypeStruct((NWORK, BATCH // NWORK), jnp.float32),
        mesh=mesh,
        compiler_params=cp,
        scratch_types=[
            pltpu.VMEM((128, 128), jnp.float32),
            pltpu.VMEM((128, 128), jnp.float32),
            pltpu.VMEM((128, 128), jnp.float32),
            pltpu.VMEM((128,), jnp.int32),
            pltpu.VMEM((128, 128), jnp.float32),
            pltpu.VMEM((128, 128), jnp.float32),
            pltpu.VMEM((128, 128), jnp.float32),
            pltpu.VMEM((128,), jnp.int32),
            pltpu.VMEM((EMB, LANES), jnp.float32),
            pltpu.VMEM((BATCH // NWORK,), jnp.float32),
            pltpu.SemaphoreType.DMA,
            pltpu.SemaphoreType.DMA,
        ],
    )

    h32, t32, r32 = (x.astype(jnp.int32) for x in (h, t, r))
    relc = jnp.concatenate([rel_w, rel_proj_w], axis=1)
    staged = extract(ent_w.T, ent_proj_w.T, h32, t32)
    scores = score(staged, relc, r32)
    return scores.reshape(BATCH)


# speedup vs baseline: 1.1312x; 1.1312x over previous
import functools

import jax
import jax.numpy as jnp
from jax import lax
from jax.experimental import pallas as pl
from jax.experimental.pallas import tpu as pltpu
from jax.experimental.pallas import tpu_sc as plsc

EMB = 64
LANES = 16
NCORES = 2
NWORK = 32
ENT = 1000000
BATCH = 16384
NQ = 2 * BATCH
NBLK = 7813
BLKS_PW = 246
EPW = BLKS_PW * 128
QCAP = 1552
SEGCAP = 192
NSUP = 16
DUMP = NQ
SROWS = 192


def _iota16():
    return lax.iota(jnp.int32, LANES)


def _bcast(vec, lane):
    idx = jnp.full((LANES,), 0, jnp.int32) + lane
    return vec.at[idx].get(mode="promise_in_bounds")


def _extract_body(ent_t, proj_t, h, t, staged,
                  scanb, qe, qp, qe2, qp2, ae, ap,
                  bufw_a, bufp_a, bufw_b, bufp_b,
                  srow, posb, scnt, sem_a, sem_b, sem_f):
    wid = lax.axis_index("s") * NCORES + lax.axis_index("c")
    b0 = wid * BLKS_PW
    e0 = b0 * 128
    e1 = jnp.minimum(e0 + EPW, ENT)
    it16 = _iota16()

    def scan_src(src_hbm, pos_base, cnt0):
        cnt = cnt0
        for p in range(BATCH // 2048):
            pltpu.sync_copy(src_hbm.at[pl.ds(p * 2048, 2048)], scanb)

            def body(i, cnt, p=p, pos_base=pos_base):
                e = scanb[pl.ds(i * LANES, LANES)]
                pos = it16 + (i * LANES + (pos_base + p * 2048))
                m = (e >= e0) & (e < e1)
                plsc.store_compressed(qe.at[pl.ds(cnt, LANES)], e, mask=m)
                plsc.store_compressed(qp.at[pl.ds(cnt, LANES)], pos, mask=m)
                cnt = cnt + plsc.all_reduce_population_count(m)[0]
                return jnp.minimum(cnt, QCAP - LANES)
            cnt = lax.fori_loop(0, 2048 // LANES, body, cnt)
        return cnt

    cntq = scan_src(h, 0, jnp.int32(0))
    cntq = scan_src(t, BATCH, cntq)

    for s in range(NSUP):
        lo = e0 + s * (16 * 128)
        hi = jnp.minimum(lo + 16 * 128, e1)

        def seg_body(i, c2, lo=lo, hi=hi, s=s):
            e = qe[pl.ds(i * LANES, LANES)]
            p = qp[pl.ds(i * LANES, LANES)]
            valid = (i * LANES + it16) < cntq
            m = valid & (e >= lo) & (e < hi)
            plsc.store_compressed(qe2.at[pl.ds(s * SEGCAP + c2, LANES)], e, mask=m)
            plsc.store_compressed(qp2.at[pl.ds(s * SEGCAP + c2, LANES)], p, mask=m)
            c2 = c2 + plsc.all_reduce_population_count(m)[0]
            return jnp.minimum(c2, SEGCAP - LANES)

        c2 = lax.fori_loop(0, QCAP // LANES, seg_body, jnp.int32(0))
        scnt[s] = c2

    blk_lim = jnp.minimum(b0 + BLKS_PW, NBLK)

    def issue(a, bufw, bufp):
        sem = sem_a if bufw is bufw_a else sem_b

        @pl.when(a + 1 < blk_lim)
        def _():
            c0 = a * 128
            pltpu.async_copy(ent_t.at[:, pl.ds(c0, 256)], bufw, sem)
            pltpu.async_copy(proj_t.at[:, pl.ds(c0, 256)], bufp, sem)

        @pl.when((a < blk_lim) & (a + 1 >= blk_lim))
        def _():
            c0 = a * 128
            pltpu.async_copy(ent_t.at[:, pl.ds(c0, 128)],
                             bufw.at[:, pl.ds(0, 128)], sem)
            pltpu.async_copy(proj_t.at[:, pl.ds(c0, 128)],
                             bufp.at[:, pl.ds(0, 128)], sem)

    def wait_set(a, bufw, bufp, sem):
        @pl.when(a + 1 < blk_lim)
        def _():
            pltpu.make_async_copy(ent_t.at[:, pl.ds(0, 256)], bufw, sem).wait()
            pltpu.make_async_copy(proj_t.at[:, pl.ds(0, 256)], bufp, sem).wait()

        @pl.when((a < blk_lim) & (a + 1 >= blk_lim))
        def _():
            pltpu.make_async_copy(ent_t.at[:, pl.ds(0, 128)],
                                  bufw.at[:, pl.ds(0, 128)], sem).wait()
            pltpu.make_async_copy(proj_t.at[:, pl.ds(0, 128)],
                                  bufp.at[:, pl.ds(0, 128)], sem).wait()

    def gather_block(b, bufw, bufp, col_base, slot):
        s = (b - b0) >> 4
        sbase = s * SEGCAP
        slim = scnt[s]

        def find(i, cb):
            e = qe2[pl.ds(sbase + i * LANES, LANES)]
            p = qp2[pl.ds(sbase + i * LANES, LANES)]
            valid = (i * LANES + it16) < slim
            m = valid & ((e >> 7) == b)
            plsc.store_compressed(ae.at[pl.ds(cb, LANES)], e, mask=m)
            plsc.store_compressed(ap.at[pl.ds(cb, LANES)], p, mask=m)
            return cb + plsc.all_reduce_population_count(m)[0]

        cb = jnp.minimum(
            lax.fori_loop(0, SEGCAP // LANES, find, jnp.int32(0)), 64)

        def one_query(i, sl):
            k16 = (i // LANES) * LANES
            lane = i - k16
            esub = ae[pl.ds(k16, LANES)]
            psub = ap[pl.ds(k16, LANES)]
            cvec = (_bcast(esub, lane) & 127) + col_base
            pvec = _bcast(psub, lane)
            sl_hi = sl // 64
            sl_lo = sl - sl_hi * 64
            plsc.store_scatter(posb, [jnp.full((LANES,), 0, jnp.int32) + sl_hi,
                                      jnp.full((LANES,), 0, jnp.int32) + sl_lo],
                               pvec, mask=it16 == 0)
            slv = jnp.full((LANES,), 0, jnp.int32) + sl
            for k in range(4):
                rows = it16 + (k * LANES)
                wv = plsc.load_gather(bufw, [rows, cvec])
                pv = plsc.load_gather(bufp, [rows, cvec])
                plsc.store_scatter(srow, [slv, rows], wv)
                plsc.store_scatter(srow, [slv, rows + EMB], pv)
            return sl + 1

        return lax.fori_loop(0, cb, one_query, slot)

    def flush64(slot):
        def do(sl):
            pltpu.async_copy(srow.at[pl.ds(0, 64)], staged.at[posb.at[0]], sem_f).wait()
            rem = sl - 64
            for k in range(4):
                posb[0, pl.ds(k * LANES, LANES)] = posb[1, pl.ds(k * LANES, LANES)]
                posb[1, pl.ds(k * LANES, LANES)] = posb[2, pl.ds(k * LANES, LANES)]

            def mv(i, _):
                src = jnp.full((LANES,), 64, jnp.int32) + i
                dst = jnp.full((LANES,), 0, jnp.int32) + i
                for k in range(8):
                    cols = _iota16() + (k * LANES)
                    v = plsc.load_gather(srow, [src, cols])
                    plsc.store_scatter(srow, [dst, cols], v)
                return 0
            lax.fori_loop(0, rem, mv, 0)
            return rem
        return lax.cond(slot >= 64, do, lambda sl: sl, slot)

    def process_pair(a, bufw, bufp, slot):
        slot = lax.cond(a < blk_lim,
                        lambda sl: gather_block(a, bufw, bufp, 0, sl),
                        lambda sl: sl, slot)
        slot = lax.cond(a + 1 < blk_lim,
                        lambda sl: gather_block(a + 1, bufw, bufp, 128, sl),
                        lambda sl: sl, slot)
        return flush64(flush64(slot))

    issue(b0, bufw_a, bufp_a)

    def quad_body(k4, slot):
        a = b0 + 4 * k4
        issue(a + 2, bufw_b, bufp_b)
        wait_set(a, bufw_a, bufp_a, sem_a)
        slot = process_pair(a, bufw_a, bufp_a, slot)
        issue(a + 4, bufw_a, bufp_a)
        wait_set(a + 2, bufw_b, bufp_b, sem_b)
        return process_pair(a + 2, bufw_b, bufp_b, slot)

    slot = lax.fori_loop(0, (BLKS_PW + 3) // 4, quad_body, jnp.int32(0))

    for k in range(4):
        lanes = it16 + (k * LANES)
        cur = posb[0, pl.ds(k * LANES, LANES)]
        posb[0, pl.ds(k * LANES, LANES)] = jnp.where(
            lanes < slot, cur, DUMP + lanes)
    pltpu.async_copy(srow.at[pl.ds(0, 64)], staged.at[posb.at[0]], sem_f).wait()


def _score_body(staged, relc, r, out,
                hbuf0, tbuf0, rbuf0, ridx0, hbuf1, tbuf1, rbuf1, ridx1,
                out_v, sem0, sem1):
    wid = lax.axis_index("s") * NCORES + lax.axis_index("c")
    rows_pw = out_v.shape[0]
    base = wid * rows_pw
    it16 = _iota16()
    nch = rows_pw // 128
    sets = [(hbuf0, tbuf0, rbuf0, ridx0, sem0),
            (hbuf1, tbuf1, rbuf1, ridx1, sem1)]

    def issue(c):
        hbuf, tbuf, rbuf, ridx, sem = sets[c % 2]
        off = base + c * 128
        pltpu.sync_copy(r.at[pl.ds(off, 128)], ridx)
        pltpu.async_copy(staged.at[pl.ds(off, 128)], hbuf, sem)
        pltpu.async_copy(staged.at[pl.ds(BATCH + off, 128)], tbuf, sem)
        pltpu.async_copy(relc.at[ridx], rbuf, sem)

    def wait(c):
        hbuf, tbuf, rbuf, ridx, sem = sets[c % 2]
        off = base + c * 128
        pltpu.make_async_copy(staged.at[pl.ds(off, 128)], hbuf, sem).wait()
        pltpu.make_async_copy(staged.at[pl.ds(off, 128)], tbuf, sem).wait()
        pltpu.make_async_copy(staged.at[pl.ds(off, 128)], rbuf, sem).wait()

    issue(0)
    for c in range(nch):
        wait(c)
        if c + 1 < nch:
            issue(c + 1)
        hbuf, tbuf, rbuf, ridx, _ = sets[c % 2]

        def group_body(g, carry, c=c, hbuf=hbuf, tbuf=tbuf, rbuf=rbuf):
            rows = it16 + g * LANES
            sh = [jnp.zeros((LANES,), jnp.float32) for _ in range(4)]
            st = [jnp.zeros((LANES,), jnp.float32) for _ in range(4)]
            for j in range(EMB):
                cj = jnp.full((LANES,), j, jnp.int32)
                he_j = plsc.load_gather(hbuf, [rows, cj])
                hp_j = plsc.load_gather(hbuf, [rows, cj + EMB])
                te_j = plsc.load_gather(tbuf, [rows, cj])
                tp_j = plsc.load_gather(tbuf, [rows, cj + EMB])
                sh[j % 4] = sh[j % 4] + he_j * hp_j
                st[j % 4] = st[j % 4] + te_j * tp_j
            a = (sh[0] + sh[1]) + (sh[2] + sh[3]) \
                - ((st[0] + st[1]) + (st[2] + st[3]))
            acc = [jnp.zeros((LANES,), jnp.float32) for _ in range(4)]
            for j in range(EMB):
                cj = jnp.full((LANES,), j, jnp.int32)
                he_j = plsc.load_gather(hbuf, [rows, cj])
                te_j = plsc.load_gather(tbuf, [rows, cj])
                re_j = plsc.load_gather(rbuf, [rows, cj])
                rp_j = plsc.load_gather(rbuf, [rows, cj + EMB])
                acc[j % 4] = acc[j % 4] + jnp.abs(he_j - te_j + re_j + a * rp_j)
            score = (acc[0] + acc[1]) + (acc[2] + acc[3])
            out_v[pl.ds(c * 128 + g * LANES, LANES)] = score
            return carry

        lax.fori_loop(0, 8, group_body, 0)

    pltpu.sync_copy(out_v, out.at[wid])


def kernel(ent_w, rel_w, ent_proj_w, rel_proj_w, h, t, r):
    mesh = plsc.VectorSubcoreMesh(core_axis_name="c", subcore_axis_name="s")
    cp = pltpu.CompilerParams(use_tc_tiling_on_sc=True,
                              needs_layout_passes=False)

    extract = pl.kernel(
        _extract_body,
        out_type=jax.ShapeDtypeStruct((NQ + 64, 128), jnp.float32),
        mesh=mesh,
        compiler_params=cp,
        scratch_types=[
            pltpu.VMEM((2048,), jnp.int32),
            pltpu.VMEM((QCAP,), jnp.int32),
            pltpu.VMEM((QCAP,), jnp.int32),
            pltpu.VMEM((NSUP * SEGCAP,), jnp.int32),
            pltpu.VMEM((NSUP * SEGCAP,), jnp.int32),
            pltpu.VMEM((80,), jnp.int32),
            pltpu.VMEM((80,), jnp.int32),
            pltpu.VMEM((EMB, 256), jnp.float32),
            pltpu.VMEM((EMB, 256), jnp.float32),
            pltpu.VMEM((EMB, 256), jnp.float32),
            pltpu.VMEM((EMB, 256), jnp.float32),
            pltpu.VMEM((SROWS, 128), jnp.float32),
            pltpu.VMEM((3, 64), jnp.int32),
            pltpu.SMEM((NSUP,), jnp.int32),
            pltpu.SemaphoreType.DMA,
            pltpu.SemaphoreType.DMA,
            pltpu.SemaphoreType.DMA,
        ],
    )

    score = pl.kernel(
        _score_body,
        out_type=jax.ShapeDtypeStruct((NWORK, BATCH // NWORK), jnp.float32),
        mesh=mesh,
        compiler_params=cp,
        scratch_types=[
            pltpu.VMEM((128, 128), jnp.float32),
            pltpu.VMEM((128, 128), jnp.float32),
            pltpu.VMEM((128, 128), jnp.float32),
            pltpu.VMEM((128,), jnp.int32),
            pltpu.VMEM((128, 128), jnp.float32),
            pltpu.VMEM((128, 128), jnp.float32),
            pltpu.VMEM((128, 128), jnp.float32),
            pltpu.VMEM((128,), jnp.int32),
            pltpu.VMEM((BATCH // NWORK,), jnp.float32),
            pltpu.SemaphoreType.DMA,
            pltpu.SemaphoreType.DMA,
        ],
    )

    h32, t32, r32 = (x.astype(jnp.int32) for x in (h, t, r))
    relc = jnp.concatenate([rel_w, rel_proj_w], axis=1)
    staged = extract(ent_w.T, ent_proj_w.T, h32, t32)
    scores = score(staged, relc, r32)
    return scores.reshape(BATCH)


# speedup vs baseline: 1.2457x; 1.1013x over previous
import functools

import jax
import jax.numpy as jnp
from jax import lax
from jax.experimental import pallas as pl
from jax.experimental.pallas import tpu as pltpu
from jax.experimental.pallas import tpu_sc as plsc

EMB = 64
LANES = 16
NCORES = 2
NWORK = 32
ENT = 1000000
BATCH = 16384
NQ = 2 * BATCH
NBLK = 7813
BLKS_PW = 246
EPW = BLKS_PW * 128
QCAP = 1552
SEGCAP = 192
NSUP = 16
DUMP = NQ
SROWS = 192


def _iota16():
    return lax.iota(jnp.int32, LANES)


def _bcast(vec, lane):
    idx = jnp.full((LANES,), 0, jnp.int32) + lane
    return vec.at[idx].get(mode="promise_in_bounds")


def _extract_body(ent_t, proj_t, h, t, staged,
                  scanb, qe, qp, qe2, qp2, ae, ap,
                  bufw_a, bufp_a, bufw_b, bufp_b,
                  srow, posb, scnt, sem_a, sem_b, sem_f):
    wid = lax.axis_index("s") * NCORES + lax.axis_index("c")
    b0 = wid * BLKS_PW
    e0 = b0 * 128
    e1 = jnp.minimum(e0 + EPW, ENT)
    it16 = _iota16()

    def scan_src(src_hbm, pos_base, cnt0):
        cnt = cnt0
        for p in range(BATCH // 2048):
            pltpu.sync_copy(src_hbm.at[pl.ds(p * 2048, 2048)], scanb)

            def body(i, cnt, p=p, pos_base=pos_base):
                e = scanb[pl.ds(i * LANES, LANES)]
                pos = it16 + (i * LANES + (pos_base + p * 2048))
                m = (e >= e0) & (e < e1)
                plsc.store_compressed(qe.at[pl.ds(cnt, LANES)], e, mask=m)
                plsc.store_compressed(qp.at[pl.ds(cnt, LANES)], pos, mask=m)
                cnt = cnt + plsc.all_reduce_population_count(m)[0]
                return jnp.minimum(cnt, QCAP - LANES)
            cnt = lax.fori_loop(0, 2048 // LANES, body, cnt)
        return cnt

    cntq = scan_src(h, 0, jnp.int32(0))
    cntq = scan_src(t, BATCH, cntq)

    for s in range(NSUP):
        lo = e0 + s * (16 * 128)
        hi = jnp.minimum(lo + 16 * 128, e1)

        def seg_body(i, c2, lo=lo, hi=hi, s=s):
            e = qe[pl.ds(i * LANES, LANES)]
            p = qp[pl.ds(i * LANES, LANES)]
            valid = (i * LANES + it16) < cntq
            m = valid & (e >= lo) & (e < hi)
            plsc.store_compressed(qe2.at[pl.ds(s * SEGCAP + c2, LANES)], e, mask=m)
            plsc.store_compressed(qp2.at[pl.ds(s * SEGCAP + c2, LANES)], p, mask=m)
            c2 = c2 + plsc.all_reduce_population_count(m)[0]
            return jnp.minimum(c2, SEGCAP - LANES)

        c2 = lax.fori_loop(0, QCAP // LANES, seg_body, jnp.int32(0))
        scnt[s] = c2

    blk_lim = jnp.minimum(b0 + BLKS_PW, NBLK)

    def issue(a, bufw, bufp):
        sem = sem_a if bufw is bufw_a else sem_b

        @pl.when(a + 1 < blk_lim)
        def _():
            c0 = a * 128
            pltpu.async_copy(ent_t.at[:, pl.ds(c0, 256)], bufw, sem)
            pltpu.async_copy(proj_t.at[:, pl.ds(c0, 256)], bufp, sem)

        @pl.when((a < blk_lim) & (a + 1 >= blk_lim))
        def _():
            c0 = a * 128
            pltpu.async_copy(ent_t.at[:, pl.ds(c0, 128)],
                             bufw.at[:, pl.ds(0, 128)], sem)
            pltpu.async_copy(proj_t.at[:, pl.ds(c0, 128)],
                             bufp.at[:, pl.ds(0, 128)], sem)

    def wait_set(a, bufw, bufp, sem):
        @pl.when(a + 1 < blk_lim)
        def _():
            pltpu.make_async_copy(ent_t.at[:, pl.ds(0, 256)], bufw, sem).wait()
            pltpu.make_async_copy(proj_t.at[:, pl.ds(0, 256)], bufp, sem).wait()

        @pl.when((a < blk_lim) & (a + 1 >= blk_lim))
        def _():
            pltpu.make_async_copy(ent_t.at[:, pl.ds(0, 128)],
                                  bufw.at[:, pl.ds(0, 128)], sem).wait()
            pltpu.make_async_copy(proj_t.at[:, pl.ds(0, 128)],
                                  bufp.at[:, pl.ds(0, 128)], sem).wait()

    def gather_block(b, bufw, bufp, col_base, slot):
        s = (b - b0) >> 4
        sbase = s * SEGCAP
        slim = scnt[s]

        def find(i, cb):
            e = qe2[pl.ds(sbase + i * LANES, LANES)]
            p = qp2[pl.ds(sbase + i * LANES, LANES)]
            valid = (i * LANES + it16) < slim
            m = valid & ((e >> 7) == b)
            plsc.store_compressed(ae.at[pl.ds(cb, LANES)], e, mask=m)
            plsc.store_compressed(ap.at[pl.ds(cb, LANES)], p, mask=m)
            return cb + plsc.all_reduce_population_count(m)[0]

        cb = jnp.minimum(
            lax.fori_loop(0, SEGCAP // LANES, find, jnp.int32(0)), 64)

        def one_query(i, sl):
            k16 = (i // LANES) * LANES
            lane = i - k16
            esub = ae[pl.ds(k16, LANES)]
            psub = ap[pl.ds(k16, LANES)]
            cvec = (_bcast(esub, lane) & 127) + col_base
            pvec = _bcast(psub, lane)
            sl_hi = sl // 64
            sl_lo = sl - sl_hi * 64
            plsc.store_scatter(posb, [jnp.full((LANES,), 0, jnp.int32) + sl_hi,
                                      jnp.full((LANES,), 0, jnp.int32) + sl_lo],
                               pvec, mask=it16 == 0)
            slv = jnp.full((LANES,), 0, jnp.int32) + sl
            dv = jnp.zeros((LANES,), jnp.float32)
            for k in range(4):
                rows = it16 + (k * LANES)
                wv = plsc.load_gather(bufw, [rows, cvec])
                pv = plsc.load_gather(bufp, [rows, cvec])
                plsc.store_scatter(srow, [slv, rows], wv)
                dv = dv + wv * pv
            dot = _bcast(plsc.cumsum(dv), 15)
            plsc.store_scatter(srow, [slv, jnp.full((LANES,), EMB, jnp.int32)],
                               dot, mask=it16 == 0)
            return sl + 1

        return lax.fori_loop(0, cb, one_query, slot)

    def flush64(slot):
        def do(sl):
            pltpu.async_copy(srow.at[pl.ds(0, 64)], staged.at[posb.at[0]], sem_f).wait()
            rem = sl - 64
            for k in range(4):
                posb[0, pl.ds(k * LANES, LANES)] = posb[1, pl.ds(k * LANES, LANES)]
                posb[1, pl.ds(k * LANES, LANES)] = posb[2, pl.ds(k * LANES, LANES)]

            def mv(i, _):
                src = jnp.full((LANES,), 64, jnp.int32) + i
                dst = jnp.full((LANES,), 0, jnp.int32) + i
                for k in range(8):
                    cols = _iota16() + (k * LANES)
                    v = plsc.load_gather(srow, [src, cols])
                    plsc.store_scatter(srow, [dst, cols], v)
                return 0
            lax.fori_loop(0, rem, mv, 0)
            return rem
        return lax.cond(slot >= 64, do, lambda sl: sl, slot)

    def process_pair(a, bufw, bufp, slot):
        slot = lax.cond(a < blk_lim,
                        lambda sl: gather_block(a, bufw, bufp, 0, sl),
                        lambda sl: sl, slot)
        slot = lax.cond(a + 1 < blk_lim,
                        lambda sl: gather_block(a + 1, bufw, bufp, 128, sl),
                        lambda sl: sl, slot)
        return flush64(flush64(slot))

    issue(b0, bufw_a, bufp_a)

    def quad_body(k4, slot):
        a = b0 + 4 * k4
        issue(a + 2, bufw_b, bufp_b)
        wait_set(a, bufw_a, bufp_a, sem_a)
        slot = process_pair(a, bufw_a, bufp_a, slot)
        issue(a + 4, bufw_a, bufp_a)
        wait_set(a + 2, bufw_b, bufp_b, sem_b)
        return process_pair(a + 2, bufw_b, bufp_b, slot)

    slot = lax.fori_loop(0, (BLKS_PW + 3) // 4, quad_body, jnp.int32(0))

    for k in range(4):
        lanes = it16 + (k * LANES)
        cur = posb[0, pl.ds(k * LANES, LANES)]
        posb[0, pl.ds(k * LANES, LANES)] = jnp.where(
            lanes < slot, cur, DUMP + lanes)
    pltpu.async_copy(srow.at[pl.ds(0, 64)], staged.at[posb.at[0]], sem_f).wait()


def _score_body(staged, relc, r, out,
                hbuf0, tbuf0, rbuf0, ridx0, hbuf1, tbuf1, rbuf1, ridx1,
                out_v, sem0, sem1):
    wid = lax.axis_index("s") * NCORES + lax.axis_index("c")
    rows_pw = out_v.shape[0]
    base = wid * rows_pw
    it16 = _iota16()
    nch = rows_pw // 128
    sets = [(hbuf0, tbuf0, rbuf0, ridx0, sem0),
            (hbuf1, tbuf1, rbuf1, ridx1, sem1)]

    def issue(c):
        hbuf, tbuf, rbuf, ridx, sem = sets[c % 2]
        off = base + c * 128
        pltpu.sync_copy(r.at[pl.ds(off, 128)], ridx)
        pltpu.async_copy(staged.at[pl.ds(off, 128)], hbuf, sem)
        pltpu.async_copy(staged.at[pl.ds(BATCH + off, 128)], tbuf, sem)
        pltpu.async_copy(relc.at[ridx], rbuf, sem)

    def wait(c):
        hbuf, tbuf, rbuf, ridx, sem = sets[c % 2]
        off = base + c * 128
        pltpu.make_async_copy(staged.at[pl.ds(off, 128)], hbuf, sem).wait()
        pltpu.make_async_copy(staged.at[pl.ds(off, 128)], tbuf, sem).wait()
        pltpu.make_async_copy(staged.at[pl.ds(off, 128)], rbuf, sem).wait()

    issue(0)
    for c in range(nch):
        wait(c)
        if c + 1 < nch:
            issue(c + 1)
        hbuf, tbuf, rbuf, ridx, _ = sets[c % 2]

        def group_body(g, carry, c=c, hbuf=hbuf, tbuf=tbuf, rbuf=rbuf):
            rows = it16 + g * LANES
            c64 = jnp.full((LANES,), EMB, jnp.int32)
            a = plsc.load_gather(hbuf, [rows, c64]) \
                - plsc.load_gather(tbuf, [rows, c64])
            acc = [jnp.zeros((LANES,), jnp.float32) for _ in range(4)]
            for j in range(EMB):
                cj = jnp.full((LANES,), j, jnp.int32)
                he_j = plsc.load_gather(hbuf, [rows, cj])
                te_j = plsc.load_gather(tbuf, [rows, cj])
                re_j = plsc.load_gather(rbuf, [rows, cj])
                rp_j = plsc.load_gather(rbuf, [rows, cj + EMB])
                acc[j % 4] = acc[j % 4] + jnp.abs(he_j - te_j + re_j + a * rp_j)
            score = (acc[0] + acc[1]) + (acc[2] + acc[3])
            out_v[pl.ds(c * 128 + g * LANES, LANES)] = score
            return carry

        lax.fori_loop(0, 8, group_body, 0)

    pltpu.sync_copy(out_v, out.at[wid])


def kernel(ent_w, rel_w, ent_proj_w, rel_proj_w, h, t, r):
    mesh = plsc.VectorSubcoreMesh(core_axis_name="c", subcore_axis_name="s")
    cp = pltpu.CompilerParams(use_tc_tiling_on_sc=True,
                              needs_layout_passes=False)

    extract = pl.kernel(
        _extract_body,
        out_type=jax.ShapeDtypeStruct((NQ + 64, 128), jnp.float32),
        mesh=mesh,
        compiler_params=cp,
        scratch_types=[
            pltpu.VMEM((2048,), jnp.int32),
            pltpu.VMEM((QCAP,), jnp.int32),
            pltpu.VMEM((QCAP,), jnp.int32),
            pltpu.VMEM((NSUP * SEGCAP,), jnp.int32),
            pltpu.VMEM((NSUP * SEGCAP,), jnp.int32),
            pltpu.VMEM((80,), jnp.int32),
            pltpu.VMEM((80,), jnp.int32),
            pltpu.VMEM((EMB, 256), jnp.float32),
            pltpu.VMEM((EMB, 256), jnp.float32),
            pltpu.VMEM((EMB, 256), jnp.float32),
            pltpu.VMEM((EMB, 256), jnp.float32),
            pltpu.VMEM((SROWS, 128), jnp.float32),
            pltpu.VMEM((3, 64), jnp.int32),
            pltpu.SMEM((NSUP,), jnp.int32),
            pltpu.SemaphoreType.DMA,
            pltpu.SemaphoreType.DMA,
            pltpu.SemaphoreType.DMA,
        ],
    )

    score = pl.kernel(
        _score_body,
        out_type=jax.ShapeDtypeStruct((NWORK, BATCH // NWORK), jnp.float32),
        mesh=mesh,
        compiler_params=cp,
        scratch_types=[
            pltpu.VMEM((128, 128), jnp.float32),
            pltpu.VMEM((128, 128), jnp.float32),
            pltpu.VMEM((128, 128), jnp.float32),
            pltpu.VMEM((128,), jnp.int32),
            pltpu.VMEM((128, 128), jnp.float32),
            pltpu.VMEM((128, 128), jnp.float32),
            pltpu.VMEM((128, 128), jnp.float32),
            pltpu.VMEM((128,), jnp.int32),
            pltpu.VMEM((BATCH // NWORK,), jnp.float32),
            pltpu.SemaphoreType.DMA,
            pltpu.SemaphoreType.DMA,
        ],
    )

    h32, t32, r32 = (x.astype(jnp.int32) for x in (h, t, r))
    relc = jnp.concatenate([rel_w, rel_proj_w], axis=1)
    staged = extract(ent_w.T, ent_proj_w.T, h32, t32)
    scores = score(staged, relc, r32)
    return scores.reshape(BATCH)
